# baseline (device time: 22938 ns/iter reference)
import jax
import jax.numpy as jnp
from jax import lax
from jax.experimental import pallas as pl
from jax.experimental.pallas import tpu as pltpu

N_DEV = 16
BLK = 128


def kernel(x, w_mat):
    k, m_per = x.shape
    k_w, n = w_mat.shape

    def body(x_ref, w_ref, out_ref, gather_ref, send_sems, recv_sems):
        my = lax.axis_index("i")

        barrier_sem = pltpu.get_barrier_semaphore()
        for r in range(1, N_DEV):
            peer = lax.rem(my + r, N_DEV)
            pl.semaphore_signal(
                barrier_sem, inc=1,
                device_id=(peer,), device_id_type=pl.DeviceIdType.MESH,
            )
        pl.semaphore_wait(barrier_sem, N_DEV - 1)

        gather_ref[0, :, :] = x_ref[pl.ds(my * BLK, BLK), :]

        rdmas = []
        for r in range(1, N_DEV):
            dst = lax.rem(my + r, N_DEV)
            rdma = pltpu.make_async_remote_copy(
                src_ref=x_ref.at[pl.ds(dst * BLK, BLK), :],
                dst_ref=gather_ref.at[r],
                send_sem=send_sems.at[r],
                recv_sem=recv_sems.at[r],
                device_id=(dst,),
                device_id_type=pl.DeviceIdType.MESH,
            )
            rdma.start()
            rdmas.append(rdma)

        out_ref[:, :] = jnp.dot(
            gather_ref[0], w_ref[pl.ds(my * BLK, BLK), :],
            preferred_element_type=jnp.float32,
        )
        for r in range(1, N_DEV):
            rdmas[r - 1].wait_recv()
            j = lax.rem(my - r + N_DEV, N_DEV)
            out_ref[:, :] += jnp.dot(
                gather_ref[r], w_ref[pl.ds(j * BLK, BLK), :],
                preferred_element_type=jnp.float32,
            )

        for r in range(1, N_DEV):
            rdmas[r - 1].wait_send()

        y = out_ref[:, :]
        out_ref[:, :] = y * lax.logistic(y)

    return pl.pallas_call(
        body,
        out_shape=jax.ShapeDtypeStruct((BLK, n), jnp.float32),
        in_specs=[
            pl.BlockSpec(memory_space=pltpu.VMEM),
            pl.BlockSpec(memory_space=pltpu.VMEM),
        ],
        out_specs=pl.BlockSpec(memory_space=pltpu.VMEM),
        scratch_shapes=[
            pltpu.VMEM((N_DEV, BLK, BLK), jnp.float32),
            pltpu.SemaphoreType.DMA((N_DEV,)),
            pltpu.SemaphoreType.DMA((N_DEV,)),
        ],
        compiler_params=pltpu.CompilerParams(collective_id=0),
    )(x, w_mat)
